# baseline (speedup 1.0000x reference)
import functools

import jax
import jax.numpy as jnp
from jax import lax
from jax.experimental import pallas as pl
from jax.experimental.pallas import tpu as pltpu
from jax.experimental.pallas import tpu_sc as plsc

BATCH = 4096
VOCAB = 100000
DIM = 64

_INFO = plsc.get_sparse_core_info()
_NC = _INFO.num_cores
_NS = _INFO.num_subcores
_L = _INFO.num_lanes
_NW = _NC * _NS

_TCOLS = (VOCAB + 127) // 128
_COLS_BASE = _TCOLS // _NW
_COLS_EXTRA = _TCOLS - _COLS_BASE * _NW
_CHUNK_COLS = 6
_CHUNK_V = _CHUNK_COLS * 128
_MAX_CHUNKS = (_COLS_BASE + 1 + _CHUNK_COLS - 1) // _CHUNK_COLS


@functools.partial(
    pl.kernel,
    mesh=plsc.VectorSubcoreMesh(core_axis_name="c", subcore_axis_name="s"),
    out_type=jax.ShapeDtypeStruct((BATCH, DIM), jnp.float32),
    scratch_types=[
        pltpu.VMEM((BATCH,), jnp.int32),
        pltpu.VMEM((BATCH,), jnp.int32),
        pltpu.VMEM((BATCH,), jnp.int32),
        pltpu.VMEM((BATCH,), jnp.int32),
        pltpu.VMEM((BATCH,), jnp.int32),
        pltpu.VMEM((DIM, _CHUNK_V), jnp.float32),
        pltpu.VMEM((DIM, _CHUNK_V), jnp.float32),
        pltpu.VMEM((_L, DIM), jnp.float32),
        pltpu.SemaphoreType.DMA,
        pltpu.SemaphoreType.DMA,
        pltpu.SemaphoreType.DMA,
    ],
    compiler_params=pltpu.CompilerParams(needs_layout_passes=False),
)
def _sc_gather(constt_hbm, idx_hbm, packed_hbm,
               idx_v, mv, mb, cv, cb, slab0, slab1, packbuf, sem, sem_s0, sem_s1):
    wid = lax.axis_index("s") * _NC + lax.axis_index("c")
    col_start = wid * _COLS_BASE + jnp.minimum(wid, _COLS_EXTRA)
    ncols = _COLS_BASE + jnp.where(wid < _COLS_EXTRA, 1, 0)
    lo = col_start * 128
    hi = jnp.minimum((col_start + ncols) * 128, VOCAB)

    lanes = lax.iota(jnp.int32, _L)
    slabs = (slab0, slab1)

    def _slab_copy(cc, sl, sm):
        vs = (col_start + cc * _CHUNK_COLS) * 128
        return pltpu.make_async_copy(
            constt_hbm.at[:, pl.ds(vs, _CHUNK_V)], sl, sm
        )

    @pl.when(0 < ncols)
    def _prime():
        _slab_copy(0, slab0, sem_s0).start()

    pltpu.sync_copy(idx_hbm, idx_v)

    def scan_body(j, m):
        for u in range(2):
            vec = idx_v[pl.ds((2 * j + u) * _L, _L)]
            msk = (vec >= lo) & (vec < hi)
            cnt = plsc.all_reduce_population_count(msk)[0]
            plsc.store_compressed(mv.at[pl.ds(m, _L)], vec, mask=msk)
            plsc.store_compressed(mb.at[pl.ds(m, _L)], (2 * j + u) * _L + lanes, mask=msk)
            m = m + cnt
        return m

    n_match = lax.fori_loop(0, BATCH // _L // 2, scan_body, 0)

    for cc in range(_MAX_CHUNKS):
        slab = slabs[cc % 2]

        @pl.when(cc * _CHUNK_COLS < ncols)
        def _chunk():
            vs = (col_start + cc * _CHUNK_COLS) * 128
            _slab_copy(cc, slab, (sem_s0, sem_s1)[cc % 2]).wait()
            if cc + 1 < _MAX_CHUNKS:
                @pl.when((cc + 1) * _CHUNK_COLS < ncols)
                def _next():
                    _slab_copy(cc + 1, slabs[(cc + 1) % 2], (sem_s0, sem_s1)[(cc + 1) % 2]).start()

            def filt_body(j, m):
                vvec = mv[pl.ds(j * _L, _L)]
                bvec = mb[pl.ds(j * _L, _L)]
                msk = ((j * _L + lanes) < n_match) & (vvec >= vs) & (vvec < vs + _CHUNK_V)
                cnt = plsc.all_reduce_population_count(msk)[0]
                plsc.store_compressed(cv.at[pl.ds(m, _L)], vvec, mask=msk)
                plsc.store_compressed(cb.at[pl.ds(m, _L)], bvec, mask=msk)
                return m + cnt

            n_ch = lax.fori_loop(0, (n_match + _L - 1) // _L, filt_body, 0)

            def grp_body(g, carry):
                vvec = cv[pl.ds(g * _L, _L)]
                bvec = cb[pl.ds(g * _L, _L)]
                vloc = jnp.clip(vvec - vs, 0, _CHUNK_V - 1)
                for l in range(_L):
                    for d0 in range(0, DIM, _L):
                        packbuf[l, pl.ds(d0, _L)] = plsc.load_gather(
                            slab,
                            [d0 + lanes, jnp.full((_L,), vloc[l], jnp.int32)],
                        )
                for l in range(_L):
                    @pl.when(g * _L + l < n_ch)
                    def _start():
                        pltpu.make_async_copy(
                            packbuf.at[pl.ds(l, 1), :],
                            packed_hbm.at[pl.ds(bvec[l], 1), :],
                            sem,
                        ).start()
                for l in range(_L):
                    @pl.when(g * _L + l < n_ch)
                    def _drain():
                        pltpu.make_async_copy(
                            packbuf.at[pl.ds(l, 1), :],
                            packed_hbm.at[pl.ds(bvec[l], 1), :],
                            sem,
                        ).wait()
                return carry

            lax.fori_loop(0, (n_ch + _L - 1) // _L, grp_body, 0)

    del n_match


_TCB = 1024


def _tc_body(packed_ref, xt_ref, out_ref):
    blk = packed_ref[...]
    eye = jnp.eye(_TCB, dtype=jnp.float32)
    blk_t = lax.dot_general(
        blk, eye,
        dimension_numbers=(((0,), (0,)), ((), ())),
        preferred_element_type=jnp.float32,
    )
    out_ref[...] = blk_t + xt_ref[...]


def _tc_finish(packed, xt):
    return pl.pallas_call(
        _tc_body,
        grid=(BATCH // _TCB,),
        in_specs=[
            pl.BlockSpec((_TCB, DIM), lambda w: (w, 0)),
            pl.BlockSpec((DIM, _TCB), lambda w: (0, w)),
        ],
        out_specs=pl.BlockSpec((DIM, _TCB), lambda w: (0, w)),
        out_shape=jax.ShapeDtypeStruct((DIM, BATCH), jnp.float32),
    )(packed, xt)


def kernel(x, const, indices):
    packed = _sc_gather(const.T, indices.astype(jnp.int32))
    outt = _tc_finish(packed, x.T)
    return outt.T


# speedup vs baseline: 1.0007x; 1.0007x over previous
import functools

import jax
import jax.numpy as jnp
from jax import lax
from jax.experimental import pallas as pl
from jax.experimental.pallas import tpu as pltpu
from jax.experimental.pallas import tpu_sc as plsc

BATCH = 4096
VOCAB = 100000
DIM = 64

_INFO = plsc.get_sparse_core_info()
_NC = _INFO.num_cores
_NS = _INFO.num_subcores
_L = _INFO.num_lanes
_NW = _NC * _NS

_TCOLS = (VOCAB + 127) // 128
_COLS_BASE = _TCOLS // _NW
_COLS_EXTRA = _TCOLS - _COLS_BASE * _NW
_CHUNK_COLS = 6
_CHUNK_V = _CHUNK_COLS * 128
_MAX_CHUNKS = (_COLS_BASE + 1 + _CHUNK_COLS - 1) // _CHUNK_COLS


@functools.partial(
    pl.kernel,
    mesh=plsc.VectorSubcoreMesh(core_axis_name="c", subcore_axis_name="s"),
    out_type=jax.ShapeDtypeStruct((BATCH, DIM), jnp.float32),
    scratch_types=[
        pltpu.VMEM((BATCH,), jnp.int32),
        pltpu.VMEM((BATCH,), jnp.int32),
        pltpu.VMEM((BATCH,), jnp.int32),
        pltpu.VMEM((BATCH,), jnp.int32),
        pltpu.VMEM((BATCH,), jnp.int32),
        pltpu.VMEM((DIM, _CHUNK_V), jnp.float32),
        pltpu.VMEM((DIM, _CHUNK_V), jnp.float32),
        pltpu.VMEM((_L, DIM), jnp.float32),
        pltpu.SemaphoreType.DMA,
        pltpu.SemaphoreType.DMA,
        pltpu.SemaphoreType.DMA,
    ],
    compiler_params=pltpu.CompilerParams(needs_layout_passes=False),
)
def _sc_gather(constt_hbm, idx_hbm, packed_hbm,
               idx_v, mv, mb, cv, cb, slab0, slab1, packbuf, sem, sem_s0, sem_s1):
    wid = lax.axis_index("s") * _NC + lax.axis_index("c")
    col_start = wid * _COLS_BASE + jnp.minimum(wid, _COLS_EXTRA)
    ncols = _COLS_BASE + jnp.where(wid < _COLS_EXTRA, 1, 0)
    lo = col_start * 128
    hi = jnp.minimum((col_start + ncols) * 128, VOCAB)

    lanes = lax.iota(jnp.int32, _L)
    slabs = (slab0, slab1)

    def _slab_copy(cc, sl, sm):
        vs = (col_start + cc * _CHUNK_COLS) * 128
        return pltpu.make_async_copy(
            constt_hbm.at[:, pl.ds(vs, _CHUNK_V)], sl, sm
        )

    @pl.when(0 < ncols)
    def _prime():
        _slab_copy(0, slab0, sem_s0).start()

    pltpu.sync_copy(idx_hbm, idx_v)

    def scan_body(j, m):
        vec = idx_v[pl.ds(j * _L, _L)]
        msk = (vec >= lo) & (vec < hi)
        cnt = plsc.all_reduce_population_count(msk)[0]
        plsc.store_compressed(mv.at[pl.ds(m, _L)], vec, mask=msk)
        plsc.store_compressed(mb.at[pl.ds(m, _L)], j * _L + lanes, mask=msk)
        return m + cnt

    n_match = lax.fori_loop(0, BATCH // _L, scan_body, 0)

    for cc in range(_MAX_CHUNKS):
        slab = slabs[cc % 2]

        @pl.when(cc * _CHUNK_COLS < ncols)
        def _chunk():
            vs = (col_start + cc * _CHUNK_COLS) * 128
            _slab_copy(cc, slab, (sem_s0, sem_s1)[cc % 2]).wait()
            if cc + 1 < _MAX_CHUNKS:
                @pl.when((cc + 1) * _CHUNK_COLS < ncols)
                def _next():
                    _slab_copy(cc + 1, slabs[(cc + 1) % 2], (sem_s0, sem_s1)[(cc + 1) % 2]).start()

            def filt_body(j, m):
                vvec = mv[pl.ds(j * _L, _L)]
                bvec = mb[pl.ds(j * _L, _L)]
                msk = ((j * _L + lanes) < n_match) & (vvec >= vs) & (vvec < vs + _CHUNK_V)
                cnt = plsc.all_reduce_population_count(msk)[0]
                plsc.store_compressed(cv.at[pl.ds(m, _L)], vvec, mask=msk)
                plsc.store_compressed(cb.at[pl.ds(m, _L)], bvec, mask=msk)
                return m + cnt

            n_ch = lax.fori_loop(0, (n_match + _L - 1) // _L, filt_body, 0)

            def grp_body(g, carry):
                vvec = cv[pl.ds(g * _L, _L)]
                bvec = cb[pl.ds(g * _L, _L)]
                vloc = jnp.clip(vvec - vs, 0, _CHUNK_V - 1)
                for l in range(_L):
                    for d0 in range(0, DIM, _L):
                        packbuf[l, pl.ds(d0, _L)] = plsc.load_gather(
                            slab,
                            [d0 + lanes, jnp.full((_L,), vloc[l], jnp.int32)],
                        )
                for l in range(_L):
                    @pl.when(g * _L + l < n_ch)
                    def _start():
                        pltpu.make_async_copy(
                            packbuf.at[pl.ds(l, 1), :],
                            packed_hbm.at[pl.ds(bvec[l], 1), :],
                            sem,
                        ).start()
                for l in range(_L):
                    @pl.when(g * _L + l < n_ch)
                    def _drain():
                        pltpu.make_async_copy(
                            packbuf.at[pl.ds(l, 1), :],
                            packed_hbm.at[pl.ds(bvec[l], 1), :],
                            sem,
                        ).wait()
                return carry

            lax.fori_loop(0, (n_ch + _L - 1) // _L, grp_body, 0)

    del n_match


_TCB = 1024


def _tc_body(packed_ref, xt_ref, out_ref):
    blk = packed_ref[...]
    eye = jnp.eye(_TCB, dtype=jnp.float32)
    blk_t = lax.dot_general(
        blk, eye,
        dimension_numbers=(((0,), (0,)), ((), ())),
        preferred_element_type=jnp.float32,
    )
    out_ref[...] = blk_t + xt_ref[...]


def _tc_finish(packed, xt):
    return pl.pallas_call(
        _tc_body,
        grid=(BATCH // _TCB,),
        in_specs=[
            pl.BlockSpec((_TCB, DIM), lambda w: (w, 0)),
            pl.BlockSpec((DIM, _TCB), lambda w: (0, w)),
        ],
        out_specs=pl.BlockSpec((DIM, _TCB), lambda w: (0, w)),
        out_shape=jax.ShapeDtypeStruct((DIM, BATCH), jnp.float32),
    )(packed, xt)


def kernel(x, const, indices):
    packed = _sc_gather(const.T, indices.astype(jnp.int32))
    outt = _tc_finish(packed, x.T)
    return outt.T


# speedup vs baseline: 1.0159x; 1.0152x over previous
import functools

import jax
import jax.numpy as jnp
from jax import lax
from jax.experimental import pallas as pl
from jax.experimental.pallas import tpu as pltpu
from jax.experimental.pallas import tpu_sc as plsc

BATCH = 4096
VOCAB = 100000
DIM = 64

_INFO = plsc.get_sparse_core_info()
_NC = _INFO.num_cores
_NS = _INFO.num_subcores
_L = _INFO.num_lanes
_NW = _NC * _NS

_TCOLS = (VOCAB + 127) // 128
_COLS_BASE = _TCOLS // _NW
_COLS_EXTRA = _TCOLS - _COLS_BASE * _NW
_CHUNK_COLS = 6
_CHUNK_V = _CHUNK_COLS * 128
_MAX_CHUNKS = (_COLS_BASE + 1 + _CHUNK_COLS - 1) // _CHUNK_COLS


@functools.partial(
    pl.kernel,
    mesh=plsc.VectorSubcoreMesh(core_axis_name="c", subcore_axis_name="s"),
    out_type=jax.ShapeDtypeStruct((BATCH, DIM), jnp.float32),
    scratch_types=[
        pltpu.VMEM((BATCH,), jnp.int32),
        pltpu.VMEM((BATCH,), jnp.int32),
        pltpu.VMEM((BATCH,), jnp.int32),
        pltpu.VMEM((BATCH,), jnp.int32),
        pltpu.VMEM((BATCH,), jnp.int32),
        pltpu.VMEM((DIM, _CHUNK_V), jnp.float32),
        pltpu.VMEM((DIM, _CHUNK_V), jnp.float32),
        pltpu.VMEM((_L, DIM), jnp.float32),
        pltpu.SemaphoreType.DMA,
        pltpu.SemaphoreType.DMA,
        pltpu.SemaphoreType.DMA,
    ],
    compiler_params=pltpu.CompilerParams(needs_layout_passes=False),
)
def _sc_gather(constt_hbm, idx_hbm, packed_hbm,
               idx_v, mv, mb, cv, cb, slab0, slab1, packbuf, sem, sem_s0, sem_s1):
    wid = lax.axis_index("s") * _NC + lax.axis_index("c")
    col_start = wid * _COLS_BASE + jnp.minimum(wid, _COLS_EXTRA)
    ncols = _COLS_BASE + jnp.where(wid < _COLS_EXTRA, 1, 0)
    lo = col_start * 128
    hi = jnp.minimum((col_start + ncols) * 128, VOCAB)

    lanes = lax.iota(jnp.int32, _L)
    slabs = (slab0, slab1)

    def _slab_copy(cc, sl, sm):
        vs = (col_start + cc * _CHUNK_COLS) * 128
        return pltpu.make_async_copy(
            constt_hbm.at[:, pl.ds(vs, _CHUNK_V)], sl, sm
        )

    pltpu.sync_copy(idx_hbm, idx_v)

    @pl.when(0 < ncols)
    def _prime():
        _slab_copy(0, slab0, sem_s0).start()

    def scan_body(j, m):
        vec = idx_v[pl.ds(j * _L, _L)]
        msk = (vec >= lo) & (vec < hi)
        cnt = plsc.all_reduce_population_count(msk)[0]
        plsc.store_compressed(mv.at[pl.ds(m, _L)], vec, mask=msk)
        plsc.store_compressed(mb.at[pl.ds(m, _L)], j * _L + lanes, mask=msk)
        return m + cnt

    n_match = lax.fori_loop(0, BATCH // _L, scan_body, 0)

    for cc in range(_MAX_CHUNKS):
        slab = slabs[cc % 2]

        @pl.when(cc * _CHUNK_COLS < ncols)
        def _chunk():
            vs = (col_start + cc * _CHUNK_COLS) * 128
            _slab_copy(cc, slab, (sem_s0, sem_s1)[cc % 2]).wait()
            if cc + 1 < _MAX_CHUNKS:
                @pl.when((cc + 1) * _CHUNK_COLS < ncols)
                def _next():
                    _slab_copy(cc + 1, slabs[(cc + 1) % 2], (sem_s0, sem_s1)[(cc + 1) % 2]).start()

            def filt_body(j, m):
                vvec = mv[pl.ds(j * _L, _L)]
                bvec = mb[pl.ds(j * _L, _L)]
                msk = ((j * _L + lanes) < n_match) & (vvec >= vs) & (vvec < vs + _CHUNK_V)
                cnt = plsc.all_reduce_population_count(msk)[0]
                plsc.store_compressed(cv.at[pl.ds(m, _L)], vvec, mask=msk)
                plsc.store_compressed(cb.at[pl.ds(m, _L)], bvec, mask=msk)
                return m + cnt

            n_ch = lax.fori_loop(0, (n_match + _L - 1) // _L, filt_body, 0)

            def grp_body(g, carry):
                vvec = cv[pl.ds(g * _L, _L)]
                bvec = cb[pl.ds(g * _L, _L)]
                vloc = jnp.clip(vvec - vs, 0, _CHUNK_V - 1)
                for l in range(_L):
                    for d0 in range(0, DIM, _L):
                        packbuf[l, pl.ds(d0, _L)] = plsc.load_gather(
                            slab,
                            [d0 + lanes, jnp.full((_L,), vloc[l], jnp.int32)],
                        )
                for l in range(_L):
                    @pl.when(g * _L + l < n_ch)
                    def _start():
                        pltpu.make_async_copy(
                            packbuf.at[pl.ds(l, 1), :],
                            packed_hbm.at[pl.ds(bvec[l], 1), :],
                            sem,
                        ).start()
                for l in range(_L):
                    @pl.when(g * _L + l < n_ch)
                    def _drain():
                        pltpu.make_async_copy(
                            packbuf.at[pl.ds(l, 1), :],
                            packed_hbm.at[pl.ds(bvec[l], 1), :],
                            sem,
                        ).wait()
                return carry

            lax.fori_loop(0, (n_ch + _L - 1) // _L, grp_body, 0)

    del n_match


_TCB = 1024


def _tc_body(packed_ref, xt_ref, out_ref):
    blk = packed_ref[...]
    eye = jnp.eye(_TCB, dtype=jnp.float32)
    blk_t = lax.dot_general(
        blk, eye,
        dimension_numbers=(((0,), (0,)), ((), ())),
        preferred_element_type=jnp.float32,
    )
    out_ref[...] = blk_t + xt_ref[...]


def _tc_finish(packed, xt):
    return pl.pallas_call(
        _tc_body,
        grid=(BATCH // _TCB,),
        in_specs=[
            pl.BlockSpec((_TCB, DIM), lambda w: (w, 0)),
            pl.BlockSpec((DIM, _TCB), lambda w: (0, w)),
        ],
        out_specs=pl.BlockSpec((DIM, _TCB), lambda w: (0, w)),
        out_shape=jax.ShapeDtypeStruct((DIM, BATCH), jnp.float32),
    )(packed, xt)


def kernel(x, const, indices):
    packed = _sc_gather(const.T, indices.astype(jnp.int32))
    outt = _tc_finish(packed, x.T)
    return outt.T


# speedup vs baseline: 1.0221x; 1.0061x over previous
import functools

import jax
import jax.numpy as jnp
from jax import lax
from jax.experimental import pallas as pl
from jax.experimental.pallas import tpu as pltpu
from jax.experimental.pallas import tpu_sc as plsc

BATCH = 4096
VOCAB = 100000
DIM = 64

_INFO = plsc.get_sparse_core_info()
_NC = _INFO.num_cores
_NS = _INFO.num_subcores
_L = _INFO.num_lanes
_NW = _NC * _NS

_TCOLS = (VOCAB + 127) // 128
_COLS_BASE = _TCOLS // _NW
_COLS_EXTRA = _TCOLS - _COLS_BASE * _NW
_CHUNK_COLS = 6
_CHUNK_V = _CHUNK_COLS * 128
_MAX_CHUNKS = (_COLS_BASE + 1 + _CHUNK_COLS - 1) // _CHUNK_COLS


@functools.partial(
    pl.kernel,
    mesh=plsc.VectorSubcoreMesh(core_axis_name="c", subcore_axis_name="s"),
    out_type=jax.ShapeDtypeStruct((BATCH, DIM), jnp.float32),
    scratch_types=[
        pltpu.VMEM((BATCH,), jnp.int32),
        pltpu.VMEM((BATCH,), jnp.int32),
        pltpu.VMEM((BATCH,), jnp.int32),
        pltpu.VMEM((BATCH,), jnp.int32),
        pltpu.VMEM((BATCH,), jnp.int32),
        pltpu.VMEM((DIM, _CHUNK_V), jnp.float32),
        pltpu.VMEM((DIM, _CHUNK_V), jnp.float32),
        pltpu.VMEM((_L, DIM), jnp.float32),
        pltpu.SemaphoreType.DMA,
        pltpu.SemaphoreType.DMA,
        pltpu.SemaphoreType.DMA,
    ],
    compiler_params=pltpu.CompilerParams(needs_layout_passes=False),
)
def _sc_gather(constt_hbm, idx_hbm, packed_hbm,
               idx_v, mv, mb, cv, cb, slab0, slab1, packbuf, sem, sem_s0, sem_s1):
    wid = lax.axis_index("s") * _NC + lax.axis_index("c")
    col_start = wid * _COLS_BASE + jnp.minimum(wid, _COLS_EXTRA)
    ncols = _COLS_BASE + jnp.where(wid < _COLS_EXTRA, 1, 0)
    lo = col_start * 128
    hi = jnp.minimum((col_start + ncols) * 128, VOCAB)

    lanes = lax.iota(jnp.int32, _L)
    slabs = (slab0, slab1)

    def _slab_copy(cc, sl, sm):
        vs = (col_start + cc * _CHUNK_COLS) * 128
        return pltpu.make_async_copy(
            constt_hbm.at[:, pl.ds(vs, _CHUNK_V)], sl, sm
        )

    pltpu.sync_copy(idx_hbm, idx_v)

    @pl.when(0 < ncols)
    def _prime():
        _slab_copy(0, slab0, sem_s0).start()

    def scan_body(j, m):
        vec = idx_v[pl.ds(j * _L, _L)]
        msk = (vec >= lo) & (vec < hi)
        cnt = plsc.all_reduce_population_count(msk)[0]
        plsc.store_compressed(mv.at[pl.ds(m, _L)], vec, mask=msk)
        plsc.store_compressed(mb.at[pl.ds(m, _L)], j * _L + lanes, mask=msk)
        return m + cnt

    n_match = lax.fori_loop(0, BATCH // _L, scan_body, 0)

    for cc in range(_MAX_CHUNKS):
        slab = slabs[cc % 2]

        @pl.when(cc * _CHUNK_COLS < ncols)
        def _chunk():
            vs = (col_start + cc * _CHUNK_COLS) * 128
            _slab_copy(cc, slab, (sem_s0, sem_s1)[cc % 2]).wait()
            if cc + 1 < _MAX_CHUNKS:
                @pl.when((cc + 1) * _CHUNK_COLS < ncols)
                def _next():
                    _slab_copy(cc + 1, slabs[(cc + 1) % 2], (sem_s0, sem_s1)[(cc + 1) % 2]).start()

            def filt_body(j, m):
                vvec = mv[pl.ds(j * _L, _L)]
                bvec = mb[pl.ds(j * _L, _L)]
                msk = ((j * _L + lanes) < n_match) & (vvec >= vs) & (vvec < vs + _CHUNK_V)
                cnt = plsc.all_reduce_population_count(msk)[0]
                plsc.store_compressed(cv.at[pl.ds(m, _L)], vvec, mask=msk)
                plsc.store_compressed(cb.at[pl.ds(m, _L)], bvec, mask=msk)
                return m + cnt

            n_ch = lax.fori_loop(0, (n_match + _L - 1) // _L, filt_body, 0)

            def grp_body(g, carry):
                vvec = cv[pl.ds(g * _L, _L)]
                bvec = cb[pl.ds(g * _L, _L)]
                vloc = jnp.clip(vvec - vs, 0, _CHUNK_V - 1)
                for l in range(_L):
                    for d0 in range(0, DIM, _L):
                        packbuf[l, pl.ds(d0, _L)] = plsc.load_gather(
                            slab,
                            [d0 + lanes, jnp.full((_L,), vloc[l], jnp.int32)],
                        )
                for l in range(_L):
                    @pl.when(g * _L + l < n_ch)
                    def _start():
                        pltpu.make_async_copy(
                            packbuf.at[pl.ds(l, 1), :],
                            packed_hbm.at[pl.ds(bvec[l], 1), :],
                            sem,
                        ).start()
                for l in range(_L):
                    @pl.when(g * _L + l < n_ch)
                    def _drain():
                        pltpu.make_async_copy(
                            packbuf.at[pl.ds(l, 1), :],
                            packed_hbm.at[pl.ds(bvec[l], 1), :],
                            sem,
                        ).wait()
                return carry

            lax.fori_loop(0, (n_ch + _L - 1) // _L, grp_body, 0)

    del n_match


_TCB = 1024


def _tc_body(packed_ref, xt_ref, out_ref):
    blk = packed_ref[...]
    out_ref[...] = blk.T + xt_ref[...]


def _tc_finish(packed, xt):
    return pl.pallas_call(
        _tc_body,
        grid=(BATCH // _TCB,),
        in_specs=[
            pl.BlockSpec((_TCB, DIM), lambda w: (w, 0)),
            pl.BlockSpec((DIM, _TCB), lambda w: (0, w)),
        ],
        out_specs=pl.BlockSpec((DIM, _TCB), lambda w: (0, w)),
        out_shape=jax.ShapeDtypeStruct((DIM, BATCH), jnp.float32),
    )(packed, xt)


def kernel(x, const, indices):
    packed = _sc_gather(const.T, indices.astype(jnp.int32))
    outt = _tc_finish(packed, x.T)
    return outt.T


# speedup vs baseline: 1.0422x; 1.0197x over previous
import functools

import jax
import jax.numpy as jnp
from jax import lax
from jax.experimental import pallas as pl
from jax.experimental.pallas import tpu as pltpu
from jax.experimental.pallas import tpu_sc as plsc

BATCH = 4096
VOCAB = 100000
DIM = 64

_INFO = plsc.get_sparse_core_info()
_NC = _INFO.num_cores
_NS = _INFO.num_subcores
_L = _INFO.num_lanes
_NW = _NC * _NS

_TCOLS = (VOCAB + 127) // 128
_COLS_BASE = _TCOLS // _NW
_COLS_EXTRA = _TCOLS - _COLS_BASE * _NW
_CHUNK_COLS = 6
_CHUNK_V = _CHUNK_COLS * 128
_MAX_CHUNKS = (_COLS_BASE + 1 + _CHUNK_COLS - 1) // _CHUNK_COLS


@functools.partial(
    pl.kernel,
    mesh=plsc.VectorSubcoreMesh(core_axis_name="c", subcore_axis_name="s"),
    out_type=jax.ShapeDtypeStruct((BATCH, DIM), jnp.float32),
    scratch_types=[
        pltpu.VMEM((BATCH,), jnp.int32),
        pltpu.VMEM((BATCH,), jnp.int32),
        pltpu.VMEM((BATCH,), jnp.int32),
        pltpu.VMEM((BATCH,), jnp.int32),
        pltpu.VMEM((BATCH,), jnp.int32),
        pltpu.VMEM((DIM, _CHUNK_V), jnp.float32),
        pltpu.VMEM((DIM, _CHUNK_V), jnp.float32),
        pltpu.VMEM((_L, DIM), jnp.float32),
        pltpu.SemaphoreType.DMA,
        pltpu.SemaphoreType.DMA,
        pltpu.SemaphoreType.DMA,
    ],
    compiler_params=pltpu.CompilerParams(needs_layout_passes=False),
)
def _sc_gather(constt_hbm, idx_hbm, packed_hbm,
               idx_v, mv, mb, cv, cb, slab0, slab1, packbuf, sem, sem_s0, sem_s1):
    wid = lax.axis_index("s") * _NC + lax.axis_index("c")
    col_start = wid * _COLS_BASE + jnp.minimum(wid, _COLS_EXTRA)
    ncols = _COLS_BASE + jnp.where(wid < _COLS_EXTRA, 1, 0)
    lo = col_start * 128
    hi = jnp.minimum((col_start + ncols) * 128, VOCAB)

    lanes = lax.iota(jnp.int32, _L)
    slabs = (slab0, slab1)

    def _slab_copy(cc, sl, sm):
        vs = (col_start + cc * _CHUNK_COLS) * 128
        return pltpu.make_async_copy(
            constt_hbm.at[:, pl.ds(vs, _CHUNK_V)], sl, sm
        )

    pltpu.sync_copy(idx_hbm, idx_v)

    @pl.when(0 < ncols)
    def _prime():
        _slab_copy(0, slab0, sem_s0).start()

    def scan_body(j, m):
        vec = idx_v[pl.ds(j * _L, _L)]
        msk = (vec >= lo) & (vec < hi)
        cnt = plsc.all_reduce_population_count(msk)[0]
        plsc.store_compressed(mv.at[pl.ds(m, _L)], vec, mask=msk)
        plsc.store_compressed(mb.at[pl.ds(m, _L)], j * _L + lanes, mask=msk)
        return m + cnt

    n_match = lax.fori_loop(0, BATCH // _L, scan_body, 0)

    for cc in range(_MAX_CHUNKS):
        slab = slabs[cc % 2]

        @pl.when(cc * _CHUNK_COLS < ncols)
        def _chunk():
            vs = (col_start + cc * _CHUNK_COLS) * 128
            _slab_copy(cc, slab, (sem_s0, sem_s1)[cc % 2]).wait()
            if cc + 1 < _MAX_CHUNKS:
                @pl.when((cc + 1) * _CHUNK_COLS < ncols)
                def _next():
                    _slab_copy(cc + 1, slabs[(cc + 1) % 2], (sem_s0, sem_s1)[(cc + 1) % 2]).start()

            def filt_body(j, m):
                vvec = mv[pl.ds(j * _L, _L)]
                bvec = mb[pl.ds(j * _L, _L)]
                msk = ((j * _L + lanes) < n_match) & (vvec >= vs) & (vvec < vs + _CHUNK_V)
                cnt = plsc.all_reduce_population_count(msk)[0]
                plsc.store_compressed(cv.at[pl.ds(m, _L)], vvec, mask=msk)
                plsc.store_compressed(cb.at[pl.ds(m, _L)], bvec, mask=msk)
                return m + cnt

            n_ch = lax.fori_loop(0, (n_match + _L - 1) // _L, filt_body, 0)

            def grp_body(g, carry):
                vvec = cv[pl.ds(g * _L, _L)]
                bvec = cb[pl.ds(g * _L, _L)]
                vloc = jnp.clip(vvec - vs, 0, _CHUNK_V - 1)
                for l in range(_L):
                    for d0 in range(0, DIM, _L):
                        packbuf[l, pl.ds(d0, _L)] = plsc.load_gather(
                            slab,
                            [d0 + lanes, jnp.full((_L,), vloc[l], jnp.int32)],
                        )
                for l in range(_L):
                    @pl.when(g * _L + l < n_ch)
                    def _start():
                        pltpu.make_async_copy(
                            packbuf.at[pl.ds(l, 1), :],
                            packed_hbm.at[pl.ds(bvec[l], 1), :],
                            sem,
                        ).start()
                for l in range(_L):
                    @pl.when(g * _L + l < n_ch)
                    def _drain():
                        pltpu.make_async_copy(
                            packbuf.at[pl.ds(l, 1), :],
                            packed_hbm.at[pl.ds(bvec[l], 1), :],
                            sem,
                        ).wait()
                return carry

            lax.fori_loop(0, (n_ch + _L - 1) // _L, grp_body, 0)

    del n_match


_TCB = 4096


def _tc_body(packed_ref, xt_ref, out_ref):
    blk = packed_ref[...]
    out_ref[...] = blk.T + xt_ref[...]


def _tc_finish(packed, xt):
    return pl.pallas_call(
        _tc_body,
        grid=(BATCH // _TCB,),
        in_specs=[
            pl.BlockSpec((_TCB, DIM), lambda w: (w, 0)),
            pl.BlockSpec((DIM, _TCB), lambda w: (0, w)),
        ],
        out_specs=pl.BlockSpec((DIM, _TCB), lambda w: (0, w)),
        out_shape=jax.ShapeDtypeStruct((DIM, BATCH), jnp.float32),
    )(packed, xt)


def kernel(x, const, indices):
    packed = _sc_gather(const.T, indices.astype(jnp.int32))
    outt = _tc_finish(packed, x.T)
    return outt.T
